# Initial kernel scaffold; baseline (speedup 1.0000x reference)
import jax
import jax.numpy as jnp
from jax.experimental import pallas as pl


def kernel(x, edge_index, batch, W1, b1, W2, b2, Wc, bc):
    raise NotImplementedError("write your pallas kernel here")


# speedup vs baseline: 7.6368x; 7.6368x over previous
import functools

import jax
import jax.numpy as jnp
from jax import lax
from jax.experimental import pallas as pl
from jax.experimental.pallas import tpu as pltpu
from jax.experimental.pallas import tpu_sc as plsc

NN = 10000
EE = 160000
G = 64
DIN = 768
DH = 128
DOUT = 20

NP = 10240
EP = 163840
CH = 128
NC = 2
NS = 16
NW = NC * NS
CPW = EP // CH // NW
RPS = NP // NS

_sc_mesh = plsc.VectorSubcoreMesh(core_axis_name="c", subcore_axis_name="s",
                                  num_cores=NC, num_subcores=NS)


@functools.partial(
    pl.kernel,
    out_type=jax.ShapeDtypeStruct((NC, NP), jnp.float32),
    mesh=_sc_mesh,
    scratch_types=[
        pltpu.VMEM((CH,), jnp.int32),
        pltpu.VMEM((CH,), jnp.float32),
        pltpu.VMEM_SHARED((NP,), jnp.float32),
    ],
)
def _deg_kernel(colp, zeros1, out, cidx_v, ones_v, deg_sh):
    c = lax.axis_index("c")
    s = lax.axis_index("s")
    w = s * NC + c
    for i in range(CH // 16):
        ones_v[pl.ds(i * 16, 16)] = jnp.ones((16,), jnp.float32)
    pltpu.sync_copy(zeros1.at[pl.ds(s * RPS, RPS)],
                    deg_sh.at[pl.ds(s * RPS, RPS)])
    plsc.subcore_barrier()

    def body(k, carry):
        base = pl.multiple_of((k * NW + w) * CH, CH)
        pltpu.sync_copy(colp.at[pl.ds(base, CH)], cidx_v)
        pltpu.sync_copy(ones_v, deg_sh.at[cidx_v], add=True)
        return carry

    lax.fori_loop(0, CPW, body, 0)
    plsc.subcore_barrier()
    pltpu.sync_copy(deg_sh.at[pl.ds(s * RPS, RPS)],
                    out.at[c, pl.ds(s * RPS, RPS)])


@functools.partial(
    pl.kernel,
    out_type=jax.ShapeDtypeStruct((NC, NP, DH), jnp.float32),
    mesh=_sc_mesh,
    scratch_types=[
        pltpu.VMEM((CH,), jnp.int32),
        pltpu.VMEM((CH,), jnp.int32),
        pltpu.VMEM((CH, DH), jnp.float32),
        pltpu.VMEM_SHARED((NP, DH), jnp.float32),
        pltpu.SemaphoreType.DMA,
    ],
)
def _agg_kernel(hs, rowp, colp, zeros2, out, ridx_v, cidx_v, rows_v, acc_sh, sem):
    c = lax.axis_index("c")
    s = lax.axis_index("s")
    w = s * NC + c
    pltpu.sync_copy(zeros2.at[pl.ds(s * RPS, RPS)],
                    acc_sh.at[pl.ds(s * RPS, RPS)])
    plsc.subcore_barrier()

    def body(k, carry):
        base = pl.multiple_of((k * NW + w) * CH, CH)
        pltpu.sync_copy(rowp.at[pl.ds(base, CH)], ridx_v)
        pltpu.sync_copy(colp.at[pl.ds(base, CH)], cidx_v)
        pltpu.async_copy(hs.at[ridx_v], rows_v, sem).wait()
        pltpu.sync_copy(rows_v, acc_sh.at[cidx_v], add=True)
        return carry

    lax.fori_loop(0, CPW, body, 0)
    plsc.subcore_barrier()
    pltpu.sync_copy(acc_sh.at[pl.ds(s * RPS, RPS)],
                    out.at[c, pl.ds(s * RPS, RPS)])


_RB = 1000


def _mm1_body(xb, dgb, W1b, hsb, dvb):
    dv = lax.rsqrt(dgb[:, 0:1] + dgb[:, 1:2] + 1.0)
    h = jnp.dot(xb[...], W1b[...], preferred_element_type=jnp.float32)
    hsb[...] = h * dv
    dvb[...] = dv


def _mm1_call(x, degp_t, W1):
    return pl.pallas_call(
        _mm1_body,
        grid=(NN // _RB,),
        in_specs=[
            pl.BlockSpec((_RB, DIN), lambda i: (i, 0)),
            pl.BlockSpec((_RB, NC), lambda i: (i, 0)),
            pl.BlockSpec((DIN, DH), lambda i: (0, 0)),
        ],
        out_specs=[
            pl.BlockSpec((_RB, DH), lambda i: (i, 0)),
            pl.BlockSpec((_RB, 1), lambda i: (i, 0)),
        ],
        out_shape=[
            jax.ShapeDtypeStruct((NN, DH), jnp.float32),
            jax.ShapeDtypeStruct((NN, 1), jnp.float32),
        ],
    )(x, degp_t, W1)


def _mm2_body(pb, hsb, dvb, b1b, W2b, outb):
    t = jnp.maximum((pb[0] + pb[1] + hsb[...]) * dvb[...] + b1b[...], 0.0)
    outb[...] = jnp.dot(t, W2b[...], preferred_element_type=jnp.float32) * dvb[...]


def _mm2_call(p1, h1s, dinv, b1, W2):
    return pl.pallas_call(
        _mm2_body,
        grid=(NN // _RB,),
        in_specs=[
            pl.BlockSpec((NC, _RB, DH), lambda i: (0, i, 0)),
            pl.BlockSpec((_RB, DH), lambda i: (i, 0)),
            pl.BlockSpec((_RB, 1), lambda i: (i, 0)),
            pl.BlockSpec((1, DH), lambda i: (0, 0)),
            pl.BlockSpec((DH, DH), lambda i: (0, 0)),
        ],
        out_specs=pl.BlockSpec((_RB, DH), lambda i: (i, 0)),
        out_shape=jax.ShapeDtypeStruct((NN, DH), jnp.float32),
    )(p1, h1s, dinv, b1, W2)


def _pool_body(pb, hsb, dvb, b2b, batchb, Wcb, bcb, outb, acc_s, acc_c):
    i = pl.program_id(0)

    @pl.when(i == 0)
    def _():
        acc_s[...] = jnp.zeros_like(acc_s)
        acc_c[...] = jnp.zeros_like(acc_c)

    h3 = jnp.maximum((pb[0] + pb[1] + hsb[...]) * dvb[...] + b2b[...], 0.0)
    oh = (batchb[...] == lax.broadcasted_iota(jnp.int32, (1, G), 1))
    oh = oh.astype(jnp.float32)
    acc_s[...] += lax.dot_general(oh, h3, (((0,), (0,)), ((), ())),
                                  preferred_element_type=jnp.float32)
    ones_col = jnp.ones((_RB, 1), jnp.float32)
    acc_c[...] += lax.dot_general(oh, ones_col, (((0,), (0,)), ((), ())),
                                  preferred_element_type=jnp.float32)

    @pl.when(i == pl.num_programs(0) - 1)
    def _():
        pooled = acc_s[...] / jnp.maximum(acc_c[...], 1.0)
        outb[...] = jnp.dot(pooled, Wcb[...],
                            preferred_element_type=jnp.float32) + bcb[...]


def _pool_call(p2, h2s, dinv, b2, batch2d, Wc, bc):
    return pl.pallas_call(
        _pool_body,
        grid=(NN // _RB,),
        in_specs=[
            pl.BlockSpec((NC, _RB, DH), lambda i: (0, i, 0)),
            pl.BlockSpec((_RB, DH), lambda i: (i, 0)),
            pl.BlockSpec((_RB, 1), lambda i: (i, 0)),
            pl.BlockSpec((1, DH), lambda i: (0, 0)),
            pl.BlockSpec((_RB, 1), lambda i: (i, 0)),
            pl.BlockSpec((DH, DOUT), lambda i: (0, 0)),
            pl.BlockSpec((1, DOUT), lambda i: (0, 0)),
        ],
        out_specs=pl.BlockSpec((G, DOUT), lambda i: (0, 0)),
        out_shape=jax.ShapeDtypeStruct((G, DOUT), jnp.float32),
        scratch_shapes=[
            pltpu.VMEM((G, DH), jnp.float32),
            pltpu.VMEM((G, 1), jnp.float32),
        ],
    )(p2, h2s, dinv, b2, batch2d, Wc, bc)


def kernel(x, edge_index, batch, W1, b1, W2, b2, Wc, bc):
    row = edge_index[0]
    col = edge_index[1]
    pad = EP - EE
    rowp = jnp.concatenate([row, jnp.zeros((pad,), row.dtype)])
    colp = jnp.concatenate([col, jnp.full((pad,), NP - 1, col.dtype)])
    zeros1 = jnp.zeros((NP,), jnp.float32)
    zeros2 = jnp.zeros((NP, DH), jnp.float32)

    degp = _deg_kernel(colp, zeros1)
    degp_t = jnp.transpose(degp)[:NN]

    h1s, dinv = _mm1_call(x, degp_t, W1)
    p1 = _agg_kernel(h1s, rowp, colp, zeros2)
    h2s = _mm2_call(p1, h1s, dinv, b1.reshape(1, -1), W2)
    p2 = _agg_kernel(h2s, rowp, colp, zeros2)
    out = _pool_call(p2, h2s, dinv, b2.reshape(1, -1),
                     batch.reshape(-1, 1), Wc, bc.reshape(1, -1))
    return out.reshape(-1)


# speedup vs baseline: 8.1279x; 1.0643x over previous
import functools

import jax
import jax.numpy as jnp
from jax import lax
from jax.experimental import pallas as pl
from jax.experimental.pallas import tpu as pltpu
from jax.experimental.pallas import tpu_sc as plsc

NN = 10000
EE = 160000
G = 64
DIN = 768
DH = 128
DOUT = 20

NP = 10240
EP = 163840
CH = 128
NC = 2
NS = 16
NW = NC * NS
CPW = EP // CH // NW
RPS = NP // NS

_sc_mesh = plsc.VectorSubcoreMesh(core_axis_name="c", subcore_axis_name="s",
                                  num_cores=NC, num_subcores=NS)


@functools.partial(
    pl.kernel,
    out_type=jax.ShapeDtypeStruct((NC, NP), jnp.float32),
    mesh=_sc_mesh,
    scratch_types=[
        pltpu.VMEM((CPW, CH), jnp.int32),
        pltpu.VMEM((CH,), jnp.float32),
        pltpu.VMEM_SHARED((NP,), jnp.float32),
    ],
)
def _deg_kernel(colp2, zeros1, out, cidx, ones_v, deg_sh):
    c = lax.axis_index("c")
    s = lax.axis_index("s")
    w = s * NC + c
    for i in range(CH // 16):
        ones_v[pl.ds(i * 16, 16)] = jnp.ones((16,), jnp.float32)
    pltpu.sync_copy(zeros1.at[pl.ds(s * RPS, RPS)],
                    deg_sh.at[pl.ds(s * RPS, RPS)])
    pltpu.sync_copy(colp2.at[pl.ds(w * CPW, CPW)], cidx)
    plsc.subcore_barrier()

    def body(k, carry):
        pltpu.sync_copy(ones_v, deg_sh.at[cidx.at[k]], add=True)
        return carry

    lax.fori_loop(0, CPW, body, 0)
    plsc.subcore_barrier()
    pltpu.sync_copy(deg_sh.at[pl.ds(s * RPS, RPS)],
                    out.at[c, pl.ds(s * RPS, RPS)])


NBUF = 2
NG = CPW // NBUF


@functools.partial(
    pl.kernel,
    out_type=jax.ShapeDtypeStruct((NC, NP, DH), jnp.float32),
    mesh=_sc_mesh,
    scratch_types=[
        pltpu.VMEM((CPW, CH), jnp.int32),
        pltpu.VMEM((CPW, CH), jnp.int32),
        pltpu.VMEM((NBUF, CH, DH), jnp.float32),
        pltpu.VMEM_SHARED((NP, DH), jnp.float32),
        pltpu.SemaphoreType.DMA,
        pltpu.SemaphoreType.DMA,
    ],
)
def _agg_kernel(hs, rowp2, colp2, zeros2, out, ridx, cidx, rows_v, acc_sh,
                sem0, sem1):
    sems = (sem0, sem1)
    c = lax.axis_index("c")
    s = lax.axis_index("s")
    w = s * NC + c
    pltpu.sync_copy(zeros2.at[pl.ds(s * RPS, RPS)],
                    acc_sh.at[pl.ds(s * RPS, RPS)])
    pltpu.sync_copy(rowp2.at[pl.ds(w * CPW, CPW)], ridx)
    pltpu.sync_copy(colp2.at[pl.ds(w * CPW, CPW)], cidx)
    plsc.subcore_barrier()

    for b in range(NBUF):
        pltpu.async_copy(hs.at[ridx.at[b]], rows_v.at[b], sems[b])

    def group(g, carry):
        for b in range(NBUF):
            k = g * NBUF + b
            pltpu.make_async_copy(hs.at[ridx.at[b]], rows_v.at[b],
                                  sems[b]).wait()
            pltpu.sync_copy(rows_v.at[b], acc_sh.at[cidx.at[k]], add=True)

            @pl.when(g < NG - 1)
            def _():
                pltpu.async_copy(hs.at[ridx.at[k + NBUF]], rows_v.at[b],
                                 sems[b])
        return carry

    lax.fori_loop(0, NG, group, 0)
    plsc.subcore_barrier()
    pltpu.sync_copy(acc_sh.at[pl.ds(s * RPS, RPS)],
                    out.at[c, pl.ds(s * RPS, RPS)])


_RB = 1000


def _mm1_body(xb, dgb, W1b, hsb, dvb):
    dv = lax.rsqrt(dgb[:, 0:1] + dgb[:, 1:2] + 1.0)
    h = jnp.dot(xb[...], W1b[...], preferred_element_type=jnp.float32)
    hsb[...] = h * dv
    dvb[...] = dv


def _mm1_call(x, degp_t, W1):
    return pl.pallas_call(
        _mm1_body,
        grid=(NN // _RB,),
        in_specs=[
            pl.BlockSpec((_RB, DIN), lambda i: (i, 0)),
            pl.BlockSpec((_RB, NC), lambda i: (i, 0)),
            pl.BlockSpec((DIN, DH), lambda i: (0, 0)),
        ],
        out_specs=[
            pl.BlockSpec((_RB, DH), lambda i: (i, 0)),
            pl.BlockSpec((_RB, 1), lambda i: (i, 0)),
        ],
        out_shape=[
            jax.ShapeDtypeStruct((NN, DH), jnp.float32),
            jax.ShapeDtypeStruct((NN, 1), jnp.float32),
        ],
    )(x, degp_t, W1)


def _mm2_body(pb, hsb, dvb, b1b, W2b, outb):
    t = jnp.maximum((pb[0] + pb[1] + hsb[...]) * dvb[...] + b1b[...], 0.0)
    outb[...] = jnp.dot(t, W2b[...], preferred_element_type=jnp.float32) * dvb[...]


def _mm2_call(p1, h1s, dinv, b1, W2):
    return pl.pallas_call(
        _mm2_body,
        grid=(NN // _RB,),
        in_specs=[
            pl.BlockSpec((NC, _RB, DH), lambda i: (0, i, 0)),
            pl.BlockSpec((_RB, DH), lambda i: (i, 0)),
            pl.BlockSpec((_RB, 1), lambda i: (i, 0)),
            pl.BlockSpec((1, DH), lambda i: (0, 0)),
            pl.BlockSpec((DH, DH), lambda i: (0, 0)),
        ],
        out_specs=pl.BlockSpec((_RB, DH), lambda i: (i, 0)),
        out_shape=jax.ShapeDtypeStruct((NN, DH), jnp.float32),
    )(p1, h1s, dinv, b1, W2)


def _pool_body(pb, hsb, dvb, b2b, batchb, Wcb, bcb, outb, acc_s, acc_c):
    i = pl.program_id(0)

    @pl.when(i == 0)
    def _():
        acc_s[...] = jnp.zeros_like(acc_s)
        acc_c[...] = jnp.zeros_like(acc_c)

    h3 = jnp.maximum((pb[0] + pb[1] + hsb[...]) * dvb[...] + b2b[...], 0.0)
    oh = (batchb[...] == lax.broadcasted_iota(jnp.int32, (1, G), 1))
    oh = oh.astype(jnp.float32)
    acc_s[...] += lax.dot_general(oh, h3, (((0,), (0,)), ((), ())),
                                  preferred_element_type=jnp.float32)
    ones_col = jnp.ones((_RB, 1), jnp.float32)
    acc_c[...] += lax.dot_general(oh, ones_col, (((0,), (0,)), ((), ())),
                                  preferred_element_type=jnp.float32)

    @pl.when(i == pl.num_programs(0) - 1)
    def _():
        pooled = acc_s[...] / jnp.maximum(acc_c[...], 1.0)
        outb[...] = jnp.dot(pooled, Wcb[...],
                            preferred_element_type=jnp.float32) + bcb[...]


def _pool_call(p2, h2s, dinv, b2, batch2d, Wc, bc):
    return pl.pallas_call(
        _pool_body,
        grid=(NN // _RB,),
        in_specs=[
            pl.BlockSpec((NC, _RB, DH), lambda i: (0, i, 0)),
            pl.BlockSpec((_RB, DH), lambda i: (i, 0)),
            pl.BlockSpec((_RB, 1), lambda i: (i, 0)),
            pl.BlockSpec((1, DH), lambda i: (0, 0)),
            pl.BlockSpec((_RB, 1), lambda i: (i, 0)),
            pl.BlockSpec((DH, DOUT), lambda i: (0, 0)),
            pl.BlockSpec((1, DOUT), lambda i: (0, 0)),
        ],
        out_specs=pl.BlockSpec((G, DOUT), lambda i: (0, 0)),
        out_shape=jax.ShapeDtypeStruct((G, DOUT), jnp.float32),
        scratch_shapes=[
            pltpu.VMEM((G, DH), jnp.float32),
            pltpu.VMEM((G, 1), jnp.float32),
        ],
    )(p2, h2s, dinv, b2, batch2d, Wc, bc)


def kernel(x, edge_index, batch, W1, b1, W2, b2, Wc, bc):
    row = edge_index[0]
    col = edge_index[1]
    pad = EP - EE
    rowp = jnp.concatenate([row, jnp.zeros((pad,), row.dtype)]).reshape(EP // CH, CH)
    colp = jnp.concatenate([col, jnp.full((pad,), NP - 1, col.dtype)]).reshape(EP // CH, CH)
    zeros1 = jnp.zeros((NP,), jnp.float32)
    zeros2 = jnp.zeros((NP, DH), jnp.float32)

    degp = _deg_kernel(colp, zeros1)
    degp_t = jnp.transpose(degp)[:NN]

    h1s, dinv = _mm1_call(x, degp_t, W1)
    p1 = _agg_kernel(h1s, rowp, colp, zeros2)
    h2s = _mm2_call(p1, h1s, dinv, b1.reshape(1, -1), W2)
    p2 = _agg_kernel(h2s, rowp, colp, zeros2)
    out = _pool_call(p2, h2s, dinv, b2.reshape(1, -1),
                     batch.reshape(-1, 1), Wc, bc.reshape(1, -1))
    return out.reshape(-1)
